# baseline (speedup 1.0000x reference)
import functools

import jax
import jax.numpy as jnp
from jax import lax
from jax.experimental import pallas as pl
from jax.experimental.pallas import tpu as pltpu
from jax.experimental.pallas import tpu_sc as plsc

N = 10000
E = 320000
G = 64
NPAD = 10240
NW = 32
EPW = 10368
KPAD = 2
EPWP = EPW + KPAD * 128
ROWS_PER_TILE = NPAD // 16

_mesh = lambda: plsc.VectorSubcoreMesh(core_axis_name="c", subcore_axis_name="s")
_SC_PARAMS = pltpu.CompilerParams(use_tc_tiling_on_sc=False)


def _zero_vmem_2d(ref, rows, cols):
    z16 = jnp.zeros((16,), jnp.float32)

    def body(i, carry):
        r = i // (cols // 16)
        k = i % (cols // 16)
        ref[r, pl.ds(k * 16, 16)] = z16
        return carry

    lax.fori_loop(0, rows * (cols // 16), body, 0)


def _deg_kernel(dst_hbm, out_hbm, dst_v, ones_v, zbuf, acc):
    c = lax.axis_index("c")
    s = lax.axis_index("s")
    w = c * 16 + s

    one16 = jnp.ones((16,), jnp.float32)

    def fill(i, carry):
        ones_v[i, :] = one16
        return carry

    lax.fori_loop(0, 128, fill, 0)
    _zero_vmem_2d(zbuf, 64, 16)

    def zslice(i, carry):
        pltpu.sync_copy(zbuf, acc.at[pl.ds(s * ROWS_PER_TILE + i * 64, 64)])
        return carry

    lax.fori_loop(0, ROWS_PER_TILE // 64, zslice, 0)

    pltpu.sync_copy(dst_hbm.at[w], dst_v)
    plsc.subcore_barrier()

    def edge_chunk(j, carry):
        pltpu.sync_copy(ones_v, acc.at[dst_v.at[j]], add=True)
        return carry

    lax.fori_loop(0, EPW // 128, edge_chunk, 0)
    plsc.subcore_barrier()

    def out_slice(i, carry):
        r0 = s * ROWS_PER_TILE + i * 64
        pltpu.sync_copy(acc.at[pl.ds(r0, 64)], zbuf)
        pltpu.sync_copy(zbuf, out_hbm.at[c, pl.ds(r0, 64)])
        return carry

    lax.fori_loop(0, ROWS_PER_TILE // 64, out_slice, 0)


def _run_deg(dst_w):
    return pl.kernel(
        _deg_kernel,
        out_type=jax.ShapeDtypeStruct((2, NPAD, 16), jnp.float32),
        mesh=_mesh(),
        compiler_params=_SC_PARAMS,
        scratch_types=[
            pltpu.VMEM((EPWP // 128, 128), jnp.int32),
            pltpu.VMEM((128, 16), jnp.float32),
            pltpu.VMEM((64, 16), jnp.float32),
            pltpu.VMEM_SHARED((NPAD, 16), jnp.float32),
        ],
    )(dst_w)


def _agg_kernel(src_hbm, dst_hbm, h_hbm, out_hbm, src_v, dst_v, rows0, rows1,
                rows2, zbuf, acc, sg0, sg1, sg2, *, d, chunk, zrows):
    c = lax.axis_index("c")
    s = lax.axis_index("s")
    w = c * 16 + s

    _zero_vmem_2d(zbuf, zrows, d)

    def zslice(i, carry):
        pltpu.sync_copy(zbuf, acc.at[pl.ds(s * ROWS_PER_TILE + i * zrows, zrows)])
        return carry

    lax.fori_loop(0, ROWS_PER_TILE // zrows, zslice, 0)

    pltpu.sync_copy(src_hbm.at[w], src_v)
    pltpu.sync_copy(dst_hbm.at[w], dst_v)
    plsc.subcore_barrier()

    def g_issue(gref, sem, j):
        pltpu.async_copy(h_hbm.at[src_v.at[j]], gref, sem)

    def g_wait(gref, sem):
        pltpu.make_async_copy(h_hbm.at[pl.ds(0, chunk)], gref, sem).wait()

    def s_sync(gref, j):
        pltpu.sync_copy(gref, acc.at[dst_v.at[j]], add=True)

    bufs = ((rows0, sg0), (rows1, sg1), (rows2, sg2))
    g_issue(rows0, sg0, 0)
    g_issue(rows1, sg1, 1)

    def outer(jj, carry):
        j = 3 * jj
        for b in range(3):
            gref, sem = bufs[b]
            nref, nsem = bufs[(b + 2) % 3]
            g_wait(gref, sem)
            g_issue(nref, nsem, j + b + 2)
            s_sync(gref, j + b)
        return carry

    lax.fori_loop(0, EPW // chunk // 3, outer, 0)
    g_wait(rows0, sg0)
    g_wait(rows1, sg1)
    plsc.subcore_barrier()

    def out_slice(i, carry):
        r0 = s * ROWS_PER_TILE + i * zrows
        pltpu.sync_copy(acc.at[pl.ds(r0, zrows)], zbuf)
        pltpu.sync_copy(zbuf, out_hbm.at[c, pl.ds(r0, zrows)])
        return carry

    lax.fori_loop(0, ROWS_PER_TILE // zrows, out_slice, 0)


def _run_agg(src_w, dst_w, h, d):
    if d <= 64:
        chunk, zrows = 128, 64
    else:
        chunk, zrows = 64, 16
    nidx = EPW // chunk + KPAD * (128 // chunk)
    return pl.kernel(
        functools.partial(_agg_kernel, d=d, chunk=chunk, zrows=zrows),
        out_type=jax.ShapeDtypeStruct((2, NPAD, d), jnp.float32),
        mesh=_mesh(),
        compiler_params=_SC_PARAMS,
        scratch_types=[
            pltpu.VMEM((nidx, chunk), jnp.int32),
            pltpu.VMEM((nidx, chunk), jnp.int32),
            pltpu.VMEM((chunk, d), jnp.float32),
            pltpu.VMEM((chunk, d), jnp.float32),
            pltpu.VMEM((chunk, d), jnp.float32),
            pltpu.VMEM((zrows, d), jnp.float32),
            pltpu.VMEM_SHARED((NPAD, d), jnp.float32),
            pltpu.SemaphoreType.DMA,
            pltpu.SemaphoreType.DMA,
            pltpu.SemaphoreType.DMA,
        ],
    )(src_w, dst_w, h)


def _mm1_body(x_ref, w_ref, h1_ref):
    h1_ref[...] = jnp.dot(x_ref[...], w_ref[...],
                          preferred_element_type=jnp.float32)


def _bn1_body(p_ref, h1s_ref, dinv_ref, a_ref, d_ref, r2_ref):
    dinv = dinv_ref[...]
    t = dinv * (p_ref[0] + p_ref[1] + h1s_ref[...])
    y = jnp.maximum(t * a_ref[...] + d_ref[...], 0.0)
    r2_ref[...] = y * dinv


def _mm2_body(p_ref, r2_ref, dinv_ref, w_ref, a_ref, d_ref, r3_ref, y2_ref):
    dinv = dinv_ref[...]
    u = dinv * (p_ref[0] + p_ref[1] + r2_ref[...])
    t = jnp.dot(u, w_ref[...], preferred_element_type=jnp.float32)
    y = jnp.maximum(t * a_ref[...] + d_ref[...], 0.0)
    y2_ref[...] = y
    r3_ref[...] = y * dinv


def _final_body(p_ref, r3_ref, y2_ref, dinv_ref, w_ref, a_ref, d_ref,
                batch_ref, fcw_ref, out_ref):
    dinv = dinv_ref[...]
    u = dinv * (p_ref[0] + p_ref[1] + r3_ref[...])
    t = jnp.dot(u, w_ref[...], preferred_element_type=jnp.float32)
    y = jnp.maximum(t * a_ref[...] + d_ref[...], 0.0)
    h = y + y2_ref[...]
    seg = lax.broadcasted_iota(jnp.int32, (G, NPAD), 0)
    mt = (seg == batch_ref[...]).astype(jnp.float32)
    sums = jnp.dot(mt, h, preferred_element_type=jnp.float32)
    cnt = jnp.sum(mt, axis=1, keepdims=True)
    pooled = sums / jnp.maximum(cnt, 1.0)
    out_ref[...] = jnp.dot(pooled, fcw_ref[...], preferred_element_type=jnp.float32)


def kernel(x, edge_index, batch, W1, b1, g1, be1, m1, v1, W2, b2, g2, be2,
           m2, v2, W3, b3, g3, be3, m3, v3, fcW, fcb):
    f32 = jnp.float32
    src = edge_index[0].astype(jnp.int32)
    dst = edge_index[1].astype(jnp.int32)
    pad = N + (jnp.arange(NW * EPW - E, dtype=jnp.int32) % (NPAD - N))
    padg = N + (jnp.arange(NW * (EPWP - EPW), dtype=jnp.int32)
                % (NPAD - N)).reshape(NW, EPWP - EPW) if EPWP > EPW else None

    def widx(a):
        flat = jnp.concatenate([a, pad]).reshape(NW, EPW)
        if EPWP > EPW:
            flat = jnp.concatenate([flat, padg], axis=1)
        return flat

    src_f = widx(src)
    dst_f = widx(dst)
    xp = jnp.pad(x.astype(f32), ((0, NPAD - N), (0, 0)))
    batch_p = jnp.pad(batch.astype(jnp.int32), (0, NPAD - N),
                      constant_values=G).reshape(1, NPAD)

    s128 = src_f.reshape(NW, EPWP // 128, 128)
    d128 = dst_f.reshape(NW, EPWP // 128, 128)
    s64 = src_f.reshape(NW, EPWP // 64, 64)
    d64 = dst_f.reshape(NW, EPWP // 64, 64)

    def fold(gq, beq, mq, vq, bq):
        aq = gq * lax.rsqrt(vq + 1e-5)
        return aq.reshape(1, -1), ((bq - mq) * aq + beq).reshape(1, -1)

    a1, d1 = fold(g1, be1, m1, v1, b1)
    a2, d2 = fold(g2, be2, m2, v2, b2)
    a3, d3 = fold(g3, be3, m3, v3, b3)

    deg_parts = _run_deg(d128)
    h1 = pl.pallas_call(
        _mm1_body,
        out_shape=jax.ShapeDtypeStruct((NPAD, 64), f32),
    )(xp, W1)
    deg = deg_parts[0, :, 0] + deg_parts[1, :, 0]
    dinv = lax.rsqrt(deg + 1.0).reshape(NPAD, 1)
    h1s = h1 * dinv

    p1 = _run_agg(s128, d128, h1s, 64)

    r2 = jnp.maximum((dinv * (p1[0] + p1[1] + h1s)) * a1 + d1, 0.0) * dinv
    p2 = _run_agg(s128, d128, r2, 64)

    r3, y2 = pl.pallas_call(
        _mm2_body,
        out_shape=[jax.ShapeDtypeStruct((NPAD, 128), f32),
                   jax.ShapeDtypeStruct((NPAD, 128), f32)],
    )(p2, r2, dinv, W2, a2, d2)

    p3 = _run_agg(s64, d64, r3, 128)

    out = pl.pallas_call(
        _final_body,
        out_shape=jax.ShapeDtypeStruct((G, 1), f32),
    )(p3, r3, y2, dinv, W3, a3, d3, batch_p, fcW)

    return (out + fcb).reshape(-1)


# speedup vs baseline: 1.0073x; 1.0073x over previous
import functools

import jax
import jax.numpy as jnp
from jax import lax
from jax.experimental import pallas as pl
from jax.experimental.pallas import tpu as pltpu
from jax.experimental.pallas import tpu_sc as plsc

N = 10000
E = 320000
G = 64
NPAD = 10240
NW = 32
EPW = 10368
KPAD = 2
EPWP = EPW + KPAD * 128
ROWS_PER_TILE = NPAD // 16

_mesh = lambda: plsc.VectorSubcoreMesh(core_axis_name="c", subcore_axis_name="s")
_SC_PARAMS = pltpu.CompilerParams(use_tc_tiling_on_sc=False)


def _zero_vmem_2d(ref, rows, cols):
    z16 = jnp.zeros((16,), jnp.float32)

    def body(i, carry):
        r = i // (cols // 16)
        k = i % (cols // 16)
        ref[r, pl.ds(k * 16, 16)] = z16
        return carry

    lax.fori_loop(0, rows * (cols // 16), body, 0)


def _deg_kernel(dst_hbm, out_hbm, dst_v, ones_v, zbuf, acc):
    c = lax.axis_index("c")
    s = lax.axis_index("s")
    w = c * 16 + s

    one16 = jnp.ones((16,), jnp.float32)

    def fill(i, carry):
        ones_v[i, :] = one16
        return carry

    lax.fori_loop(0, 128, fill, 0)
    _zero_vmem_2d(zbuf, 64, 16)

    def zslice(i, carry):
        pltpu.sync_copy(zbuf, acc.at[pl.ds(s * ROWS_PER_TILE + i * 64, 64)])
        return carry

    lax.fori_loop(0, ROWS_PER_TILE // 64, zslice, 0)

    pltpu.sync_copy(dst_hbm.at[w], dst_v)
    plsc.subcore_barrier()

    def edge_chunk(j, carry):
        pltpu.sync_copy(ones_v, acc.at[dst_v.at[j]], add=True)
        return carry

    lax.fori_loop(0, EPW // 128, edge_chunk, 0)
    plsc.subcore_barrier()

    def out_slice(i, carry):
        r0 = s * ROWS_PER_TILE + i * 64
        pltpu.sync_copy(acc.at[pl.ds(r0, 64)], zbuf)
        pltpu.sync_copy(zbuf, out_hbm.at[c, pl.ds(r0, 64)])
        return carry

    lax.fori_loop(0, ROWS_PER_TILE // 64, out_slice, 0)


def _run_deg(dst_w):
    return pl.kernel(
        _deg_kernel,
        out_type=jax.ShapeDtypeStruct((2, NPAD, 16), jnp.float32),
        mesh=_mesh(),
        compiler_params=_SC_PARAMS,
        scratch_types=[
            pltpu.VMEM((EPWP // 128, 128), jnp.int32),
            pltpu.VMEM((128, 16), jnp.float32),
            pltpu.VMEM((64, 16), jnp.float32),
            pltpu.VMEM_SHARED((NPAD, 16), jnp.float32),
        ],
    )(dst_w)


def _agg_kernel(src_hbm, dst_hbm, h_hbm, out_hbm, src_v, dst_v, rows0, rows1,
                rows2, zbuf, acc, sg0, sg1, sg2, *, d, chunk, zrows):
    c = lax.axis_index("c")
    s = lax.axis_index("s")
    w = c * 16 + s

    _zero_vmem_2d(zbuf, zrows, d)

    def zslice(i, carry):
        pltpu.sync_copy(zbuf, acc.at[pl.ds(s * ROWS_PER_TILE + i * zrows, zrows)])
        return carry

    lax.fori_loop(0, ROWS_PER_TILE // zrows, zslice, 0)

    pltpu.sync_copy(src_hbm.at[w], src_v)
    pltpu.sync_copy(dst_hbm.at[w], dst_v)
    plsc.subcore_barrier()

    def g_issue(gref, sem, j):
        pltpu.async_copy(h_hbm.at[src_v.at[j]], gref, sem)

    def g_wait(gref, sem):
        pltpu.make_async_copy(h_hbm.at[pl.ds(0, chunk)], gref, sem).wait()

    def s_sync(gref, j):
        pltpu.sync_copy(gref, acc.at[dst_v.at[j]], add=True)

    bufs = ((rows0, sg0), (rows1, sg1), (rows2, sg2))
    g_issue(rows0, sg0, 0)
    g_issue(rows1, sg1, 1)

    def outer(jj, carry):
        j = 3 * jj
        for b in range(3):
            gref, sem = bufs[b]
            nref, nsem = bufs[(b + 2) % 3]
            g_wait(gref, sem)
            g_issue(nref, nsem, j + b + 2)
            s_sync(gref, j + b)
        return carry

    lax.fori_loop(0, EPW // chunk // 3, outer, 0)
    g_wait(rows0, sg0)
    g_wait(rows1, sg1)
    plsc.subcore_barrier()

    def out_slice(i, carry):
        r0 = s * ROWS_PER_TILE + i * zrows
        pltpu.sync_copy(acc.at[pl.ds(r0, zrows)], zbuf)
        pltpu.sync_copy(zbuf, out_hbm.at[c, pl.ds(r0, zrows)])
        return carry

    lax.fori_loop(0, ROWS_PER_TILE // zrows, out_slice, 0)


def _run_agg(src_w, dst_w, h, d):
    if d <= 64:
        chunk, zrows = 128, 64
    else:
        chunk, zrows = 64, 16
    nidx = EPW // chunk + KPAD * (128 // chunk)
    return pl.kernel(
        functools.partial(_agg_kernel, d=d, chunk=chunk, zrows=zrows),
        out_type=jax.ShapeDtypeStruct((2, NPAD, d), jnp.float32),
        mesh=_mesh(),
        compiler_params=_SC_PARAMS,
        scratch_types=[
            pltpu.VMEM((nidx, chunk), jnp.int32),
            pltpu.VMEM((nidx, chunk), jnp.int32),
            pltpu.VMEM((chunk, d), jnp.float32),
            pltpu.VMEM((chunk, d), jnp.float32),
            pltpu.VMEM((chunk, d), jnp.float32),
            pltpu.VMEM((zrows, d), jnp.float32),
            pltpu.VMEM_SHARED((NPAD, d), jnp.float32),
            pltpu.SemaphoreType.DMA,
            pltpu.SemaphoreType.DMA,
            pltpu.SemaphoreType.DMA,
        ],
    )(src_w, dst_w, h)


def _mm1_body(x_ref, w_ref, h1_ref):
    h1_ref[...] = jnp.dot(x_ref[...], w_ref[...],
                          preferred_element_type=jnp.float32)


def _bn1_body(p_ref, h1s_ref, dinv_ref, a_ref, d_ref, r2_ref):
    dinv = dinv_ref[...]
    t = dinv * (p_ref[0] + p_ref[1] + h1s_ref[...])
    y = jnp.maximum(t * a_ref[...] + d_ref[...], 0.0)
    r2_ref[...] = y * dinv


def _mm2_body(p_ref, r2_ref, dinv_ref, w_ref, a_ref, d_ref, r3_ref, y2_ref):
    dinv = dinv_ref[...]
    u = dinv * (p_ref[0] + p_ref[1] + r2_ref[...])
    t = jnp.dot(u, w_ref[...], preferred_element_type=jnp.float32)
    y = jnp.maximum(t * a_ref[...] + d_ref[...], 0.0)
    y2_ref[...] = y
    r3_ref[...] = y * dinv


def _final_body(p_ref, r3_ref, y2_ref, dinv_ref, w_ref, a_ref, d_ref,
                batch_ref, fcw_ref, out_ref):
    dinv = dinv_ref[...]
    u = dinv * (p_ref[0] + p_ref[1] + r3_ref[...])
    t = jnp.dot(u, w_ref[...], preferred_element_type=jnp.float32)
    y = jnp.maximum(t * a_ref[...] + d_ref[...], 0.0)
    h = y + y2_ref[...]
    seg = lax.broadcasted_iota(jnp.int32, (G, NPAD), 0)
    mt = (seg == batch_ref[...]).astype(jnp.float32)
    sums = jnp.dot(mt, h, preferred_element_type=jnp.float32)
    cnt = jnp.sum(mt, axis=1, keepdims=True)
    pooled = sums / jnp.maximum(cnt, 1.0)
    out_ref[...] = jnp.dot(pooled, fcw_ref[...], preferred_element_type=jnp.float32)


def kernel(x, edge_index, batch, W1, b1, g1, be1, m1, v1, W2, b2, g2, be2,
           m2, v2, W3, b3, g3, be3, m3, v3, fcW, fcb):
    f32 = jnp.float32
    src = edge_index[0].astype(jnp.int32)
    dst = edge_index[1].astype(jnp.int32)
    pad = N + (jnp.arange(NW * EPW - E, dtype=jnp.int32) % (NPAD - N))
    padg = N + (jnp.arange(NW * (EPWP - EPW), dtype=jnp.int32)
                % (NPAD - N)).reshape(NW, EPWP - EPW) if EPWP > EPW else None

    def widx(a):
        flat = jnp.concatenate([a, pad]).reshape(NW, EPW)
        if EPWP > EPW:
            flat = jnp.concatenate([flat, padg], axis=1)
        return flat

    src_f = widx(src)
    dst_f = widx(dst)
    xp = jnp.pad(x.astype(f32), ((0, NPAD - N), (0, 0)))
    batch_p = jnp.pad(batch.astype(jnp.int32), (0, NPAD - N),
                      constant_values=G).reshape(1, NPAD)

    s128 = src_f.reshape(NW, EPWP // 128, 128)
    d128 = dst_f.reshape(NW, EPWP // 128, 128)
    s64 = src_f.reshape(NW, EPWP // 64, 64)
    d64 = dst_f.reshape(NW, EPWP // 64, 64)

    def fold(gq, beq, mq, vq, bq):
        aq = gq * lax.rsqrt(vq + 1e-5)
        return aq.reshape(1, -1), ((bq - mq) * aq + beq).reshape(1, -1)

    a1, d1 = fold(g1, be1, m1, v1, b1)
    a2, d2 = fold(g2, be2, m2, v2, b2)
    a3, d3 = fold(g3, be3, m3, v3, b3)

    deg_parts = _run_deg(d128)
    h1 = pl.pallas_call(
        _mm1_body,
        out_shape=jax.ShapeDtypeStruct((NPAD, 64), f32),
    )(xp, W1)
    deg = deg_parts[0, :, 0] + deg_parts[1, :, 0]
    dinv = lax.rsqrt(deg + 1.0).reshape(NPAD, 1)
    h1s = h1 * dinv

    p1 = _run_agg(s128, d128, h1s, 64)

    r2 = pl.pallas_call(
        _bn1_body,
        out_shape=jax.ShapeDtypeStruct((NPAD, 64), f32),
    )(p1, h1s, dinv, a1, d1)
    p2 = _run_agg(s128, d128, r2, 64)

    r3, y2 = pl.pallas_call(
        _mm2_body,
        out_shape=[jax.ShapeDtypeStruct((NPAD, 128), f32),
                   jax.ShapeDtypeStruct((NPAD, 128), f32)],
    )(p2, r2, dinv, W2, a2, d2)

    p3 = _run_agg(s64, d64, r3, 128)

    out = pl.pallas_call(
        _final_body,
        out_shape=jax.ShapeDtypeStruct((G, 1), f32),
    )(p3, r3, y2, dinv, W3, a3, d3, batch_p, fcW)

    return (out + fcb).reshape(-1)
